# baseline (speedup 1.0000x reference)
import functools

import jax
import jax.numpy as jnp
from jax import lax
from jax.experimental import pallas as pl
from jax.experimental.pallas import tpu as pltpu
from jax.experimental.pallas import tpu_sc as plsc

DIM = 768
SEQ = 512
BATCH = 256
OUT_DIM = 30
PAD_OUT = 32

NC = 2
NS = 16
L = 16
NW = NC * NS
ROWS_PER_W = BATCH // NW
G = 32
NBUF = 4
CH = SEQ // G
T = ROWS_PER_W * CH
JV = DIM // L


def _sc_forward(x_flat, table, wg, bg):
    mesh = plsc.VectorSubcoreMesh(core_axis_name="c", subcore_axis_name="s")

    @functools.partial(
        pl.kernel,
        out_type=jax.ShapeDtypeStruct((BATCH, PAD_OUT), jnp.float32),
        mesh=mesh,
        compiler_params=pltpu.CompilerParams(needs_layout_passes=False),
        scratch_types=[
            pltpu.VMEM((ROWS_PER_W * SEQ,), jnp.int32),
            pltpu.VMEM((NBUF, G, DIM), jnp.float32),
            pltpu.VMEM((ROWS_PER_W, DIM), jnp.float32),
            pltpu.VMEM((PAD_OUT,), jnp.float32),
            pltpu.VMEM((ROWS_PER_W, PAD_OUT), jnp.float32),
            pltpu.VMEM((L,), jnp.float32),
            pltpu.SemaphoreType.DMA,
            pltpu.SemaphoreType.DMA,
            pltpu.SemaphoreType.DMA,
            pltpu.SemaphoreType.DMA,
        ],
    )
    def fwd(x_hbm, table_hbm, wg_hbm, bg_hbm, out_hbm,
            idx_v, buf_v, acc_v, bg_v, ob_v, red_v, gs0, gs1, gs2, gs3):
        cid = lax.axis_index("c")
        sid = lax.axis_index("s")
        base = (cid * NS + sid) * ROWS_PER_W
        pltpu.sync_copy(x_hbm.at[pl.ds(base * SEQ, ROWS_PER_W * SEQ)], idx_v)

        zeros = jnp.zeros((L,), jnp.float32)

        @pl.loop(0, ROWS_PER_W)
        def _(r):
            for j in range(JV):
                acc_v[r, pl.ds(j * L, L)] = zeros

        gsems = (gs0, gs1, gs2, gs3)

        def start_gather(t, bi):
            pltpu.async_copy(
                table_hbm.at[idx_v.at[pl.ds(t * G, G)]],
                buf_v.at[bi],
                gsems[bi],
            )

        def wait_gather(bi):
            pltpu.make_async_copy(
                table_hbm.at[pl.ds(0, G)], buf_v.at[bi], gsems[bi]
            ).wait()

        for i in range(NBUF):
            start_gather(i, i)

        @pl.loop(0, T, step=NBUF)
        def _(t0):
            for bi in range(NBUF):
                t = t0 + bi
                wait_gather(bi)
                r = t // CH

                def body(g, carry):
                    return tuple(
                        c + buf_v[bi, g, pl.ds(j * L, L)]
                        for j, c in enumerate(carry)
                    )

                fin = plsc.parallel_loop(
                    0, G, unroll=2, carry=(zeros,) * JV
                )(body)
                for j in range(JV):
                    plsc.addupdate(acc_v.at[r, pl.ds(j * L, L)], fin[j])

                @pl.when(t + NBUF < T)
                def _():
                    start_gather(t + NBUF, bi)

        pltpu.sync_copy(wg_hbm, buf_v.at[0])
        pltpu.sync_copy(bg_hbm, bg_v)
        lane = lax.iota(jnp.int32, L)
        shuf = tuple(
            jnp.bitwise_and(lane + sh, L - 1) for sh in (8, 4, 2, 1)
        )
        inv = 1.0 / SEQ

        def lane_reduce(v, op):
            for idx in shuf:
                red_v[pl.ds(0, L)] = v
                v = op(v, plsc.load_gather(red_v, [idx]))
            return v

        @pl.loop(0, ROWS_PER_W)
        def _(r):
            def obody(o, carry):
                g0, g1 = carry

                def jbody(j, a):
                    return a + (
                        acc_v[r, pl.ds(j * L, L)]
                        * buf_v[0, o, pl.ds(j * L, L)]
                    )

                part = plsc.parallel_loop(0, JV, unroll=4, carry=zeros)(jbody)
                s = lane_reduce(part, jnp.add) * inv
                g0 = jnp.where((o < L) & (lane == o), s, g0)
                g1 = jnp.where((o >= L) & (lane == o - L), s, g1)
                return (g0, g1)

            g0, g1 = pl.loop(0, OUT_DIM, init_carry=(zeros, zeros))(obody)
            l0 = g0 + bg_v[pl.ds(0, L)]
            l1 = g1 + bg_v[pl.ds(L, L)]
            m = lane_reduce(jnp.maximum(l0, l1), jnp.maximum)
            e0 = jnp.exp(l0 - m)
            e1 = jnp.exp(l1 - m)
            tot = lane_reduce(e0 + e1, jnp.add)
            ob_v[r, pl.ds(0, L)] = e0 / tot
            ob_v[r, pl.ds(L, L)] = e1 / tot

        pltpu.sync_copy(ob_v, out_hbm.at[pl.ds(base, ROWS_PER_W)])

    return fwd(x_flat, table, wg, bg)


@jax.jit
def kernel(x, table, W, b):
    wg = jnp.pad(W.T, ((0, PAD_OUT - OUT_DIM), (0, 0)))
    bg = jnp.concatenate(
        [b, jnp.full((PAD_OUT - OUT_DIM,), -1e30, jnp.float32)]
    )
    probs = _sc_forward(x.reshape(-1), table, wg, bg)
    return probs[:, :OUT_DIM]


# speedup vs baseline: 1.7696x; 1.7696x over previous
import functools

import jax
import jax.numpy as jnp
from jax import lax
from jax.experimental import pallas as pl
from jax.experimental.pallas import tpu as pltpu
from jax.experimental.pallas import tpu_sc as plsc

DIM = 768
SEQ = 512
BATCH = 256
OUT_DIM = 30
PAD_OUT = 128

NC = 2
NS = 16
L = 16
NW = NC * NS
ROWS_PER_W = BATCH // NW
G = 32
NBUF = 4
CH = SEQ // G
T = ROWS_PER_W * CH
JV = DIM // L


def _sc_pool(x_flat, table):
    mesh = plsc.VectorSubcoreMesh(core_axis_name="c", subcore_axis_name="s")

    @functools.partial(
        pl.kernel,
        out_type=jax.ShapeDtypeStruct((BATCH, DIM), jnp.float32),
        mesh=mesh,
        scratch_types=[
            pltpu.VMEM((ROWS_PER_W * SEQ,), jnp.int32),
            pltpu.VMEM((NBUF, G, DIM), jnp.float32),
            pltpu.VMEM((ROWS_PER_W, DIM), jnp.float32),
            pltpu.SemaphoreType.DMA,
            pltpu.SemaphoreType.DMA,
            pltpu.SemaphoreType.DMA,
            pltpu.SemaphoreType.DMA,
        ],
    )
    def pool(x_hbm, table_hbm, out_hbm, idx_v, buf_v, acc_v, gs0, gs1, gs2, gs3):
        cid = lax.axis_index("c")
        sid = lax.axis_index("s")
        base = (cid * NS + sid) * ROWS_PER_W
        pltpu.sync_copy(x_hbm.at[pl.ds(base * SEQ, ROWS_PER_W * SEQ)], idx_v)

        zeros = jnp.zeros((L,), jnp.float32)

        @pl.loop(0, ROWS_PER_W)
        def _(r):
            for j in range(JV):
                acc_v[r, pl.ds(j * L, L)] = zeros

        gsems = (gs0, gs1, gs2, gs3)

        def start_gather(t, bi):
            pltpu.async_copy(
                table_hbm.at[idx_v.at[pl.ds(t * G, G)]],
                buf_v.at[bi],
                gsems[bi],
            )

        def wait_gather(bi):
            pltpu.make_async_copy(
                table_hbm.at[pl.ds(0, G)], buf_v.at[bi], gsems[bi]
            ).wait()

        for i in range(NBUF):
            start_gather(i, i)

        @pl.loop(0, T, step=NBUF)
        def _(t0):
            for bi in range(NBUF):
                t = t0 + bi
                wait_gather(bi)
                r = t // CH

                def body(g, carry):
                    return tuple(
                        c + buf_v[bi, g, pl.ds(j * L, L)]
                        for j, c in enumerate(carry)
                    )

                fin = plsc.parallel_loop(
                    0, G, unroll=2, carry=(zeros,) * JV
                )(body)
                for j in range(JV):
                    plsc.addupdate(acc_v.at[r, pl.ds(j * L, L)], fin[j])

                @pl.when(t + NBUF < T)
                def _():
                    start_gather(t + NBUF, bi)

        pltpu.sync_copy(acc_v, out_hbm.at[pl.ds(base, ROWS_PER_W)])

    return pool(x_flat, table)


def _tc_head(sums, w, b2):
    def body(s_ref, w_ref, b_ref, o_ref):
        pooled = s_ref[...] * (1.0 / SEQ)
        logits = (
            jnp.dot(pooled, w_ref[...], preferred_element_type=jnp.float32)
            + b_ref[...]
        )
        m = jnp.max(logits, axis=1, keepdims=True)
        e = jnp.exp(logits - m)
        o_ref[...] = e / jnp.sum(e, axis=1, keepdims=True)

    return pl.pallas_call(
        body,
        out_shape=jax.ShapeDtypeStruct((BATCH, OUT_DIM), jnp.float32),
    )(sums, w, b2)


@jax.jit
def kernel(x, table, W, b):
    sums = _sc_pool(x.reshape(-1), table)
    return _tc_head(sums, W, b.reshape(1, OUT_DIM))
